# baseline (speedup 1.0000x reference)
import functools

import jax
import jax.numpy as jnp
from jax.experimental import pallas as pl

_D = 128
_BLK = 1024


def _rows(n):
    return (n + _BLK - 1) // _BLK


def _row_spec(width=_D):
    if width == 0:
        return pl.BlockSpec((_BLK,), lambda i: (i,))
    return pl.BlockSpec((_BLK, width), lambda i: (i, 0))


def _full_spec(shape):
    return pl.BlockSpec(shape, lambda i: tuple(0 for _ in shape))


def _emb2_body(t_ref, w1_ref, b1_ref, w2_ref, b2_ref, o_ref):
    t = t_ref[...]
    h = jnp.maximum(t[:, None] * w1_ref[...] + b1_ref[...], 0.0)
    o_ref[...] = jnp.dot(h, w2_ref[...], preferred_element_type=jnp.float32) + b2_ref[...]


def _bond_init_body(oh_ref, t_ref, w1_ref, b1_ref, w2_ref, b2_ref, w3_ref, b3_ref,
                    dw1_ref, db1_ref, dw2_ref, db2_ref, o_ref):
    h = jnp.maximum(jnp.dot(oh_ref[...], w1_ref[...], preferred_element_type=jnp.float32) + b1_ref[...], 0.0)
    h = jnp.maximum(jnp.dot(h, w2_ref[...], preferred_element_type=jnp.float32) + b2_ref[...], 0.0)
    y = jnp.dot(h, w3_ref[...], preferred_element_type=jnp.float32) + b3_ref[...]
    t = t_ref[...]
    g = jnp.maximum(t[:, None] * dw1_ref[...] + db1_ref[...], 0.0)
    o_ref[...] = y + jnp.dot(g, dw2_ref[...], preferred_element_type=jnp.float32) + db2_ref[...]


def _layer_body(a_ref, g_ref, w1a_ref, w1b_ref, b1_ref, w2_ref, b2_ref, w3_ref, b3_ref, o_ref):
    a = a_ref[...]
    h = (jnp.dot(a, w1a_ref[...], preferred_element_type=jnp.float32)
         + jnp.dot(g_ref[...], w1b_ref[...], preferred_element_type=jnp.float32)
         + b1_ref[...])
    h = jnp.maximum(h, 0.0)
    h = jnp.maximum(jnp.dot(h, w2_ref[...], preferred_element_type=jnp.float32) + b2_ref[...], 0.0)
    o_ref[...] = a + jnp.dot(h, w3_ref[...], preferred_element_type=jnp.float32) + b3_ref[...]


def _head_body(n, x_ref, t_ref, w1_ref, b1_ref, w2_ref, b2_ref, w3_ref, b3_ref, o_ref):
    h = jnp.maximum(jnp.dot(x_ref[...], w1_ref[...], preferred_element_type=jnp.float32) + b1_ref[...], 0.0)
    h = jnp.maximum(jnp.dot(h, w2_ref[...], preferred_element_type=jnp.float32) + b2_ref[...], 0.0)
    pred = jnp.dot(h, w3_ref[...], preferred_element_type=jnp.float32) + b3_ref[...]
    d = pred - t_ref[...][:, None]
    a = jnp.abs(d)
    hub = jnp.where(a < 1.0, 0.5 * d * d, a - 0.5)
    col = jax.lax.broadcasted_iota(jnp.int32, hub.shape, 1)
    row = (jax.lax.broadcasted_iota(jnp.int32, hub.shape, 0)
           + pl.program_id(0) * _BLK)
    hub = jnp.where((col == 0) & (row < n), hub, 0.0)
    o_ref[...] = jnp.sum(hub, axis=0, keepdims=True)[None]


def _emb2(t, p):
    n = t.shape[0]
    w1 = p[0]['W'][0]
    w2 = p[1]['W']
    return pl.pallas_call(
        _emb2_body,
        grid=(_rows(n),),
        in_specs=[_row_spec(0), _full_spec((_D,)), _full_spec((_D,)),
                  _full_spec((_D, _D)), _full_spec((_D,))],
        out_specs=_row_spec(),
        out_shape=jax.ShapeDtypeStruct((n, _D), jnp.float32),
    )(t, w1, p[0]['b'], w2, p[1]['b'])


def _bond_init(oh, t, p3, p2):
    n = oh.shape[0]
    w1 = jnp.zeros((_D, _D), jnp.float32).at[: p3[0]['W'].shape[0]].set(p3[0]['W'])
    return pl.pallas_call(
        _bond_init_body,
        grid=(_rows(n),),
        in_specs=[_row_spec(), _row_spec(0),
                  _full_spec((_D, _D)), _full_spec((_D,)),
                  _full_spec((_D, _D)), _full_spec((_D,)),
                  _full_spec((_D, _D)), _full_spec((_D,)),
                  _full_spec((_D,)), _full_spec((_D,)),
                  _full_spec((_D, _D)), _full_spec((_D,))],
        out_specs=_row_spec(),
        out_shape=jax.ShapeDtypeStruct((n, _D), jnp.float32),
    )(oh, t, w1, p3[0]['b'], p3[1]['W'], p3[1]['b'], p3[2]['W'], p3[2]['b'],
      p2[0]['W'][0], p2[0]['b'], p2[1]['W'], p2[1]['b'])


def _layer_mlp(a, g, p):
    n = a.shape[0]
    w1a = p[0]['W'][:_D]
    w1b = p[0]['W'][_D:]
    return pl.pallas_call(
        _layer_body,
        grid=(_rows(n),),
        in_specs=[_row_spec(), _row_spec(),
                  _full_spec((_D, _D)), _full_spec((_D, _D)), _full_spec((_D,)),
                  _full_spec((_D, _D)), _full_spec((_D,)),
                  _full_spec((_D, _D)), _full_spec((_D,))],
        out_specs=_row_spec(),
        out_shape=jax.ShapeDtypeStruct((n, _D), jnp.float32),
    )(a, g, w1a, w1b, p[0]['b'], p[1]['W'], p[1]['b'], p[2]['W'], p[2]['b'])


def _head_loss(x, t, p):
    n = x.shape[0]
    w3 = jnp.zeros((_D, _D), jnp.float32).at[:, :1].set(p[2]['W'])
    b3 = jnp.zeros((_D,), jnp.float32).at[0].set(p[2]['b'][0])
    partials = pl.pallas_call(
        functools.partial(_head_body, n),
        grid=(_rows(n),),
        in_specs=[_row_spec(), _row_spec(0),
                  _full_spec((_D, _D)), _full_spec((_D,)),
                  _full_spec((_D, _D)), _full_spec((_D,)),
                  _full_spec((_D, _D)), _full_spec((_D,))],
        out_specs=pl.BlockSpec((1, 1, _D), lambda i: (i, 0, 0)),
        out_shape=jax.ShapeDtypeStruct((_rows(n), 1, _D), jnp.float32),
    )(x, t, p[0]['W'], p[0]['b'], p[1]['W'], p[1]['b'], w3, b3)
    return jnp.sum(partials) / n


def kernel(AtomBondGraph_edges, BondAngleGraph_edges, AngleDihedralGraph_edges,
           pos, x, bond_attr, bond_lengths, bond_angles, dihedral_angles,
           num_atoms, num_bonds, num_angles, num_graphs, atom_batch, params):
    del AtomBondGraph_edges, pos, x, num_atoms, num_bonds, num_angles, num_graphs, atom_batch
    BA = BondAngleGraph_edges
    AD = AngleDihedralGraph_edges
    n_bonds = bond_lengths.shape[0]
    n_angles = bond_angles.shape[0]

    vocab = [7, 5, 2]
    offs = [0, 7, 12]
    oh = jnp.zeros((n_bonds, _D), jnp.float32)
    for i, (v, o) in enumerate(zip(vocab, offs)):
        oh = oh + (jax.lax.broadcasted_iota(jnp.int32, (n_bonds, _D), 1)
                   == (bond_attr[:, i] + o)[:, None]).astype(jnp.float32)

    bond_h = _bond_init(oh, bond_lengths, params['bond_init'], params['dis_emb'])
    angle_h = _emb2(bond_angles, params['angle_emb'])
    dih_h = _emb2(dihedral_angles, params['dihedral_emb'])

    for lp in params['layers']:
        msg_ad = jnp.take(angle_h, AD[:, 0], axis=0) + dih_h
        agg_a = jax.ops.segment_sum(msg_ad, AD[:, 1], num_segments=n_angles)
        angle_h = _layer_mlp(angle_h, agg_a, lp['angle_mlp'])
        msg_ba = jnp.take(bond_h, BA[:, 0], axis=0) + angle_h
        agg_b = jax.ops.segment_sum(msg_ba, BA[:, 1], num_segments=n_bonds)
        bond_h = _layer_mlp(bond_h, agg_b, lp['bond_mlp'])

    loss = _head_loss(bond_h, bond_lengths, params['Blr_mlp'])
    loss = loss + _head_loss(angle_h, bond_angles, params['Bar_mlp'])
    loss = loss + _head_loss(dih_h, dihedral_angles, params['Dar_mlp'])
    return loss
